# baseline (speedup 1.0000x reference)
import jax
import jax.numpy as jnp
from jax.experimental import pallas as pl
from jax.experimental.pallas import tpu as pltpu

UST_DIM = 8


def _fused_kernel(x_ref, o_ref, scale_ref):
    F = x_ref.shape[1]

    @pl.when(pl.program_id(0) == 0)
    def _compute_scale():
        qi = jax.lax.broadcasted_iota(jnp.int32, (F, F), 0)
        pj = jax.lax.broadcasted_iota(jnp.int32, (F, F), 1)
        diff = (qi - pj).astype(jnp.float32)
        dists = jnp.float32(UST_DIM) * (diff * diff)
        idx = jnp.argmin(dists, axis=1)
        scale_ref[...] = ((idx.astype(jnp.float32) + 1.0) / jnp.float32(F))[None, :]

    o_ref[...] = x_ref[...] * scale_ref[...]


def kernel(inputs):
    B, F = inputs.shape
    BLK = 1024
    out = pl.pallas_call(
        _fused_kernel,
        grid=(B // BLK,),
        in_specs=[pl.BlockSpec((BLK, F), lambda i: (i, 0))],
        out_specs=pl.BlockSpec((BLK, F), lambda i: (i, 0)),
        out_shape=jax.ShapeDtypeStruct((B, F), inputs.dtype),
        scratch_shapes=[pltpu.VMEM((1, F), jnp.float32)],
        compiler_params=pltpu.CompilerParams(
            dimension_semantics=("arbitrary",),
        ),
    )(inputs)
    return out


# speedup vs baseline: 1.0421x; 1.0421x over previous
import jax
import jax.numpy as jnp
from jax.experimental import pallas as pl
from jax.experimental.pallas import tpu as pltpu

UST_DIM = 8


def _fused_kernel(x_ref, o_ref, scale_ref):
    F = x_ref.shape[1]

    @pl.when(pl.program_id(0) == 0)
    def _compute_scale():
        qi = jax.lax.broadcasted_iota(jnp.int32, (F, F), 0)
        pj = jax.lax.broadcasted_iota(jnp.int32, (F, F), 1)
        diff = (qi - pj).astype(jnp.float32)
        dists = jnp.float32(UST_DIM) * (diff * diff)
        idx = jnp.argmin(dists, axis=1)
        scale_ref[...] = ((idx.astype(jnp.float32) + 1.0) / jnp.float32(F))[None, :]

    o_ref[...] = x_ref[...] * scale_ref[...]


def kernel(inputs):
    B, F = inputs.shape
    BLK = 2048
    out = pl.pallas_call(
        _fused_kernel,
        grid=(B // BLK,),
        in_specs=[pl.BlockSpec((BLK, F), lambda i: (i, 0))],
        out_specs=pl.BlockSpec((BLK, F), lambda i: (i, 0)),
        out_shape=jax.ShapeDtypeStruct((B, F), inputs.dtype),
        scratch_shapes=[pltpu.VMEM((1, F), jnp.float32)],
        compiler_params=pltpu.CompilerParams(
            dimension_semantics=("arbitrary",),
        ),
    )(inputs)
    return out
